# Initial kernel scaffold; baseline (speedup 1.0000x reference)
import jax
import jax.numpy as jnp
from jax.experimental import pallas as pl


def kernel(x1, edge_index1, batch1, x2, edge_index2, W1, b1, W2, b2, fc1_W, fc1_b, ln1_g, ln1_b, fc2_W, fc2_b, ln2_g, ln2_b, fc3_W, fc3_b):
    raise NotImplementedError("write your pallas kernel here")


# speedup vs baseline: 27.3598x; 27.3598x over previous
import functools
import jax
import jax.numpy as jnp
from jax import lax
from jax.experimental import pallas as pl
from jax.experimental.pallas import tpu as pltpu
from jax.experimental.pallas import tpu_sc as plsc

N1, N2 = 10000, 400
NUM_GRAPHS = 64
SORT_K = 30
HID = 128

N1P, N2P = 10240, 512
E1, E2 = 640000, 6400
E1P, E2P = 655360, 8192
R1, R2 = E1P // 128, E2P // 128
NT = 16
ROWS1, ROWS2 = R1 // NT, R2 // NT
WIN1, WIN2 = 16, 4
C1G, C2G = N1P // NT, N2P // NT

_f32 = jnp.float32
_i32 = jnp.int32


def _rsqrt16(d):
    g = 1.0 / d
    for _ in range(18):
        g = g * (1.5 - 0.5 * d * g * g)
    return g


def _sc_body(src1, dst1, x1h, src2, dst2, x2h,
             p1o, q1o, p2o, q2o,
             sw, dw, ua, ub, ta, tb, c0, cdv, cy, cyp, cyq, zz,
             s1d, s1s, s1p, s1q, s1y, s1yp, s1yq,
             s2d, s2s, s2p, s2q, s2y, s2yp, s2yq):
    t = lax.axis_index("s")

    def fill(ref, n, val):
        def bd(i, _):
            ref[pl.ds(i * 16, 16)] = jnp.full((16,), val, _f32)
            return 0
        lax.fori_loop(0, n // 16, bd, 0)

    fill(zz, C1G, 0.0)

    def pipeline(srcf, dstr, xh, po, qo,
                 spd, sps, spp, spq, spy, spyp, spyq,
                 n_pad, rows_pt, win_rows, chunk):
        n_win = rows_pt // win_rows
        wedges = win_rows * 128
        groups = chunk // 16
        o = t * chunk

        for acc in (spd, sps, spp, spq):
            pltpu.sync_copy(zz.at[pl.ds(0, chunk)], acc.at[pl.ds(o, chunk)])
        plsc.subcore_barrier()

        fill(ua, wedges, 1.0)

        def degwin(w, _):
            r0 = t * rows_pt + w * win_rows
            pltpu.sync_copy(dstr.at[pl.ds(r0, win_rows)],
                            dw.at[pl.ds(0, win_rows)])

            def sc(cc, _):
                pltpu.sync_copy(ua.at[pl.ds(cc * 128, 128)],
                                spd.at[dw.at[cc]], add=True)
                return 0
            lax.fori_loop(0, win_rows, sc, 0)
            return 0
        lax.fori_loop(0, n_win, degwin, 0)
        plsc.subcore_barrier()

        pltpu.sync_copy(spd.at[pl.ds(o, chunk)], c0.at[pl.ds(0, chunk)])
        pltpu.sync_copy(xh.at[pl.ds(o, chunk)], cy.at[pl.ds(0, chunk)])

        def t1(i, _):
            s = pl.ds(i * 16, 16)
            r = _rsqrt16(c0[s] + 1.0)
            cdv[s] = r
            cy[s] = r * cy[s]
            return 0
        lax.fori_loop(0, groups, t1, 0)
        pltpu.sync_copy(cy.at[pl.ds(0, chunk)], spy.at[pl.ds(o, chunk)])
        plsc.subcore_barrier()

        pltpu.sync_copy(spy, ta.at[pl.ds(0, n_pad)])

        def win2(w, _):
            r0 = t * rows_pt + w * win_rows
            pltpu.sync_copy(srcf.at[pl.ds(r0 * 128, wedges)],
                            sw.at[pl.ds(0, wedges)])
            pltpu.sync_copy(dstr.at[pl.ds(r0, win_rows)],
                            dw.at[pl.ds(0, win_rows)])

            def g(j, _):
                s = pl.ds(j * 16, 16)
                ua[s] = plsc.load_gather(ta, [sw[s]])
                return 0
            lax.fori_loop(0, wedges // 16, g, 0)

            def sc(cc, _):
                pltpu.sync_copy(ua.at[pl.ds(cc * 128, 128)],
                                sps.at[dw.at[cc]], add=True)
                return 0
            lax.fori_loop(0, win_rows, sc, 0)
            return 0
        lax.fori_loop(0, n_win, win2, 0)
        plsc.subcore_barrier()

        pltpu.sync_copy(sps.at[pl.ds(o, chunk)], c0.at[pl.ds(0, chunk)])

        def t3(i, _):
            s = pl.ds(i * 16, 16)
            agg = cdv[s] * (c0[s] + cy[s])
            cyp[s] = cdv[s] * jnp.maximum(agg, 0.0)
            cyq[s] = cdv[s] * jnp.maximum(-agg, 0.0)
            return 0
        lax.fori_loop(0, groups, t3, 0)
        pltpu.sync_copy(cyp.at[pl.ds(0, chunk)], spyp.at[pl.ds(o, chunk)])
        pltpu.sync_copy(cyq.at[pl.ds(0, chunk)], spyq.at[pl.ds(o, chunk)])
        plsc.subcore_barrier()

        pltpu.sync_copy(spyp, ta.at[pl.ds(0, n_pad)])
        pltpu.sync_copy(spyq, tb.at[pl.ds(0, n_pad)])

        def win3(w, _):
            r0 = t * rows_pt + w * win_rows
            pltpu.sync_copy(srcf.at[pl.ds(r0 * 128, wedges)],
                            sw.at[pl.ds(0, wedges)])
            pltpu.sync_copy(dstr.at[pl.ds(r0, win_rows)],
                            dw.at[pl.ds(0, win_rows)])

            def g(j, _):
                s = pl.ds(j * 16, 16)
                idx = sw[s]
                ua[s] = plsc.load_gather(ta, [idx])
                ub[s] = plsc.load_gather(tb, [idx])
                return 0
            lax.fori_loop(0, wedges // 16, g, 0)

            def sc(cc, _):
                pltpu.sync_copy(ua.at[pl.ds(cc * 128, 128)],
                                spp.at[dw.at[cc]], add=True)
                pltpu.sync_copy(ub.at[pl.ds(cc * 128, 128)],
                                spq.at[dw.at[cc]], add=True)
                return 0
            lax.fori_loop(0, win_rows, sc, 0)
            return 0
        lax.fori_loop(0, n_win, win3, 0)
        plsc.subcore_barrier()

        pltpu.sync_copy(spp.at[pl.ds(o, chunk)], c0.at[pl.ds(0, chunk)])
        pltpu.sync_copy(spq.at[pl.ds(o, chunk)], cy.at[pl.ds(0, chunk)])

        def t5(i, _):
            s = pl.ds(i * 16, 16)
            cyp[s] = cdv[s] * (c0[s] + cyp[s])
            cyq[s] = cdv[s] * (cy[s] + cyq[s])
            return 0
        lax.fori_loop(0, groups, t5, 0)
        pltpu.sync_copy(cyp.at[pl.ds(0, chunk)], po.at[pl.ds(o, chunk)])
        pltpu.sync_copy(cyq.at[pl.ds(0, chunk)], qo.at[pl.ds(o, chunk)])
        plsc.subcore_barrier()

    pipeline(src1, dst1, x1h, p1o, q1o,
             s1d, s1s, s1p, s1q, s1y, s1yp, s1yq,
             N1P, ROWS1, WIN1, C1G)
    pipeline(src2, dst2, x2h, p2o, q2o,
             s2d, s2s, s2p, s2q, s2y, s2yp, s2yq,
             N2P, ROWS2, WIN2, C2G)


_sc_mesh = plsc.VectorSubcoreMesh(core_axis_name="c", subcore_axis_name="s",
                                  num_cores=1)

_sc_call = functools.partial(
    pl.kernel,
    out_type=(jax.ShapeDtypeStruct((N1P,), _f32),
              jax.ShapeDtypeStruct((N1P,), _f32),
              jax.ShapeDtypeStruct((N2P,), _f32),
              jax.ShapeDtypeStruct((N2P,), _f32)),
    mesh=_sc_mesh,
    compiler_params=pltpu.CompilerParams(needs_layout_passes=False),
    scratch_types=(
        pltpu.VMEM((WIN1 * 128,), _i32),
        pltpu.VMEM((WIN1, 128), _i32),
        pltpu.VMEM((WIN1 * 128,), _f32),
        pltpu.VMEM((WIN1 * 128,), _f32),
        pltpu.VMEM((N1P,), _f32),
        pltpu.VMEM((N1P,), _f32),
        pltpu.VMEM((C1G,), _f32),
        pltpu.VMEM((C1G,), _f32),
        pltpu.VMEM((C1G,), _f32),
        pltpu.VMEM((C1G,), _f32),
        pltpu.VMEM((C1G,), _f32),
        pltpu.VMEM((C1G,), _f32),
        pltpu.VMEM_SHARED((N1P,), _f32),
        pltpu.VMEM_SHARED((N1P,), _f32),
        pltpu.VMEM_SHARED((N1P,), _f32),
        pltpu.VMEM_SHARED((N1P,), _f32),
        pltpu.VMEM_SHARED((N1P,), _f32),
        pltpu.VMEM_SHARED((N1P,), _f32),
        pltpu.VMEM_SHARED((N1P,), _f32),
        pltpu.VMEM_SHARED((N2P,), _f32),
        pltpu.VMEM_SHARED((N2P,), _f32),
        pltpu.VMEM_SHARED((N2P,), _f32),
        pltpu.VMEM_SHARED((N2P,), _f32),
        pltpu.VMEM_SHARED((N2P,), _f32),
        pltpu.VMEM_SHARED((N2P,), _f32),
        pltpu.VMEM_SHARED((N2P,), _f32),
    ),
)(_sc_body)


NBLK = N1P // 256
NSLOT = SORT_K * NUM_GRAPHS
NSLOTP = 2048


def _tc_keys_body(w1, w2, b2, p1, q1, p2, q2, keys_o, out2_o, ab_o):
    w = w1[...]
    w2v = w2[...]
    b2v = b2[...]
    a = jnp.dot(jnp.maximum(w, 0.0), w2v, preferred_element_type=_f32)
    b = jnp.dot(jnp.maximum(-w, 0.0), w2v, preferred_element_type=_f32)
    ab_o[...] = jnp.concatenate([a, b, jnp.zeros((6, HID), _f32)], axis=0)
    out2 = jnp.maximum(p2[...] * a + q2[...] * b + b2v, 0.0)
    out2_o[...] = out2
    c = out2[N2 - 1:N2, :]
    n2c = jnp.sum(c * c)
    for i in range(NBLK):
        pb = p1[i * 256:(i + 1) * 256, :]
        qb = q1[i * 256:(i + 1) * 256, :]
        o1 = jnp.maximum(pb * a + qb * b + b2v, 0.0)
        n1 = jnp.sum(o1 * o1, axis=1, keepdims=True)
        dc = jnp.sum(o1 * c, axis=1, keepdims=True)
        keys_o[i * 256:(i + 1) * 256, :] = jnp.sqrt(
            jnp.maximum(n1 + n2c - 2.0 * dc, 1e-12))


def _tc_keys(w1, w2, b2, p1c, q1c, p2c, q2c):
    return pl.pallas_call(
        _tc_keys_body,
        out_shape=(jax.ShapeDtypeStruct((N1P, 1), _f32),
                   jax.ShapeDtypeStruct((N2P, HID), _f32),
                   jax.ShapeDtypeStruct((8, HID), _f32)),
    )(w1, w2, b2, p1c, q1c, p2c, q2c)


def _tc_rank_body(keysc, batchc, p1c, q1c, keysr, batchr, sel_o):
    ib = pl.program_id(0)
    iota_s = lax.broadcasted_iota(_i32, (1, NSLOTP), 1)
    iota_c = lax.broadcasted_iota(_i32, (256, 1), 0)
    iota_r = lax.broadcasted_iota(_i32, (1, 256), 1)

    ki = keysc[...]
    bi = batchc[...]
    ii = ib * 256 + iota_c
    blo_i = jnp.min(bi)
    bhi_i = jnp.max(bi)

    rank = jnp.zeros((256, 1), _i32)
    for jb in range(NBLK):
        kj = keysr[jb:jb + 1, :]
        bj = batchr[jb:jb + 1, :]
        jj = jb * 256 + iota_r
        pred = (jnp.min(bj) <= bhi_i) & (jnp.max(bj) >= blo_i)

        def _acc(r, kj=kj, bj=bj, jj=jj):
            same = bi == bj
            beats = (kj > ki) | ((kj == ki) & (jj < ii))
            return r + jnp.sum(jnp.where(same & beats, 1, 0), axis=1,
                               keepdims=True)

        rank = lax.cond(pred, _acc, lambda r: r, rank)

    valid = (rank < SORT_K) & (bi < NUM_GRAPHS)
    m = jnp.where(valid, rank * NUM_GRAPHS + bi, NSLOTP - 1)
    onehot = (m == iota_s).astype(_f32)
    dP = jnp.sum(onehot * p1c[...], axis=0, keepdims=True)
    dQ = jnp.sum(onehot * q1c[...], axis=0, keepdims=True)
    dV = jnp.sum(onehot, axis=0, keepdims=True)
    upd = jnp.concatenate([dP, dQ, dV, jnp.zeros((5, NSLOTP), _f32)], axis=0)

    @pl.when(ib == 0)
    def _():
        sel_o[...] = upd

    @pl.when(ib > 0)
    def _():
        sel_o[...] = sel_o[...] + upd


def _tc_rank(keysc, batchc, p1c, q1c, keysr, batchr):
    blk = pl.BlockSpec((256, 1), lambda i: (i, 0))
    return pl.pallas_call(
        _tc_rank_body,
        grid=(NBLK,),
        in_specs=[blk, blk, blk, blk,
                  pl.BlockSpec((NBLK, 256), lambda i: (0, 0)),
                  pl.BlockSpec((NBLK, 256), lambda i: (0, 0))],
        out_specs=pl.BlockSpec((8, NSLOTP), lambda i: (0, 0)),
        out_shape=jax.ShapeDtypeStruct((8, NSLOTP), _f32),
    )(keysc, batchc, p1c, q1c, keysr, batchr)


def _tc_dense_body(ps, qs, vs, out2, ab, b2, fc1w, fc1b, ln1g, ln1b,
                   fc2w, fc2b, ln2g, ln2b, fc3w, fc3b, out_o):
    abv = ab[...]
    a = abv[0:1, :]
    b = abv[1:2, :]
    b2v = b2[...]
    osel = jnp.maximum(ps[...] * a + qs[...] * b + b2v, 0.0)
    o2 = out2[...]
    g = lax.dot_general(osel, o2, (((1,), (1,)), ((), ())),
                        preferred_element_type=_f32)
    ns = jnp.sum(osel * osel, axis=1, keepdims=True)
    n2 = jnp.sum(o2 * o2, axis=1, keepdims=True)
    n2r = lax.dot_general(jnp.ones((1, 1), _f32), n2,
                          (((1,), (1,)), ((), ())),
                          preferred_element_type=_f32)
    drow = jnp.sqrt(jnp.maximum(ns + n2r - 2.0 * g, 1e-12)) * vs[...]

    h = jnp.zeros((NUM_GRAPHS, HID), _f32) + fc1b[...]
    for r in range(SORT_K):
        h = h + jnp.dot(drow[r * NUM_GRAPHS:(r + 1) * NUM_GRAPHS, :],
                        fc1w[r * N2P:(r + 1) * N2P, :],
                        preferred_element_type=_f32)

    def ln(x, gam, bet):
        mu = jnp.mean(x, axis=-1, keepdims=True)
        var = jnp.mean((x - mu) ** 2, axis=-1, keepdims=True)
        return (x - mu) * lax.rsqrt(var + 1e-5) * gam[...] + bet[...]

    h = jnp.maximum(ln(h, ln1g, ln1b), 0.0)
    h = jnp.dot(h, fc2w[...], preferred_element_type=_f32) + fc2b[...]
    h = jnp.maximum(ln(h, ln2g, ln2b), 0.0)
    h = jnp.dot(h, fc3w[...], preferred_element_type=_f32) + fc3b[...]
    out_o[...] = 1.0 / (1.0 + jnp.exp(-h))


def _tc_dense(ps, qs, vs, out2, ab, b2, fc1w, fc1b, ln1g, ln1b,
              fc2w, fc2b, ln2g, ln2b, fc3w, fc3b):
    return pl.pallas_call(
        _tc_dense_body,
        out_shape=jax.ShapeDtypeStruct((NUM_GRAPHS, 1), _f32),
    )(ps, qs, vs, out2, ab, b2, fc1w, fc1b, ln1g, ln1b,
      fc2w, fc2b, ln2g, ln2b, fc3w, fc3b)


@jax.jit
def kernel(x1, edge_index1, batch1, x2, edge_index2, W1, b1, W2, b2,
           fc1_W, fc1_b, ln1_g, ln1_b, fc2_W, fc2_b, ln2_g, ln2_b,
           fc3_W, fc3_b):
    pad1 = jnp.full((E1P - E1,), N1P - 1, _i32)
    src1 = jnp.concatenate([edge_index1[0].astype(_i32), pad1])
    dst1 = jnp.concatenate([edge_index1[1].astype(_i32), pad1]).reshape(R1, 128)
    pad2 = jnp.full((E2P - E2,), N2P - 1, _i32)
    src2 = jnp.concatenate([edge_index2[0].astype(_i32), pad2])
    dst2 = jnp.concatenate([edge_index2[1].astype(_i32), pad2]).reshape(R2, 128)
    x1p = jnp.pad(x1[:, 0], (0, N1P - N1))
    x2p = jnp.pad(x2[:, 0], (0, N2P - N2))
    batchp = jnp.pad(batch1.astype(_i32), (0, N1P - N1),
                     constant_values=NUM_GRAPHS)

    fc1wp = jnp.pad(fc1_W.reshape(SORT_K, N2, HID),
                    ((0, 0), (0, N2P - N2), (0, 0))).reshape(SORT_K * N2P, HID)

    p1h, q1h, p2h, q2h = _sc_call(src1, dst1, x1p, src2, dst2, x2p)

    keysc, out2, ab = _tc_keys(W1, W2, b2[None, :],
                               p1h[:, None], q1h[:, None],
                               p2h[:, None], q2h[:, None])
    keysr = keysc[:, 0].reshape(NBLK, 256)
    sel = _tc_rank(keysc, batchp[:, None], p1h[:, None], q1h[:, None],
                   keysr, batchp.reshape(NBLK, 256))
    ps = sel[0, 0:NSLOT, None]
    qs = sel[1, 0:NSLOT, None]
    vs = sel[2, 0:NSLOT, None]
    out = _tc_dense(ps, qs, vs, out2, ab, b2[None, :], fc1wp,
                    fc1_b[None, :], ln1_g[None, :], ln1_b[None, :],
                    fc2_W, fc2_b[None, :], ln2_g[None, :], ln2_b[None, :],
                    fc3_W, fc3_b[None, :])
    return out


# speedup vs baseline: 27.7505x; 1.0143x over previous
import functools
import jax
import jax.numpy as jnp
from jax import lax
from jax.experimental import pallas as pl
from jax.experimental.pallas import tpu as pltpu
from jax.experimental.pallas import tpu_sc as plsc

N1, N2 = 10000, 400
NUM_GRAPHS = 64
SORT_K = 30
HID = 128

N1P, N2P = 10240, 512
E1, E2 = 640000, 6400
E1P, E2P = 655360, 8192
R1, R2 = E1P // 128, E2P // 128
NT = 16
ROWS1, ROWS2 = R1 // NT, R2 // NT
WIN1, WIN2 = 16, 4
C1G, C2G = N1P // NT, N2P // NT

_f32 = jnp.float32
_i32 = jnp.int32


def _rsqrt16(d):
    g = 1.0 / d
    for _ in range(18):
        g = g * (1.5 - 0.5 * d * g * g)
    return g


def _sc_body(src1, dst1, x1h, src2, dst2, x2h,
             p1o, q1o, p2o, q2o,
             sw, dw, ua, ub, ta, tb, c0, cdv, cy, cyp, cyq, zz,
             s1d, s1s, s1p, s1q, s1y, s1yp, s1yq,
             s2d, s2s, s2p, s2q, s2y, s2yp, s2yq):
    t = lax.axis_index("s")

    def fill(ref, n, val):
        def bd(i, _):
            ref[pl.ds(i * 16, 16)] = jnp.full((16,), val, _f32)
            return 0
        lax.fori_loop(0, n // 16, bd, 0)

    fill(zz, C1G, 0.0)

    def pipeline(srcf, dstr, xh, po, qo,
                 spd, sps, spp, spq, spy, spyp, spyq,
                 n_pad, rows_pt, win_rows, chunk):
        n_win = rows_pt // win_rows
        wedges = win_rows * 128
        groups = chunk // 16
        o = t * chunk

        for acc in (spd, sps, spp, spq):
            pltpu.sync_copy(zz.at[pl.ds(0, chunk)], acc.at[pl.ds(o, chunk)])
        plsc.subcore_barrier()

        fill(ua, wedges, 1.0)

        def degwin(w, _):
            r0 = t * rows_pt + w * win_rows
            pltpu.sync_copy(dstr.at[pl.ds(r0, win_rows)],
                            dw.at[pl.ds(0, win_rows)])

            def sc(cc, _):
                pltpu.sync_copy(ua.at[pl.ds(cc * 128, 128)],
                                spd.at[dw.at[cc]], add=True)
                return 0
            lax.fori_loop(0, win_rows, sc, 0)
            return 0
        lax.fori_loop(0, n_win, degwin, 0)
        plsc.subcore_barrier()

        pltpu.sync_copy(spd.at[pl.ds(o, chunk)], c0.at[pl.ds(0, chunk)])
        pltpu.sync_copy(xh.at[pl.ds(o, chunk)], cy.at[pl.ds(0, chunk)])

        def t1(i, _):
            s = pl.ds(i * 16, 16)
            r = _rsqrt16(c0[s] + 1.0)
            cdv[s] = r
            cy[s] = r * cy[s]
            return 0
        lax.fori_loop(0, groups, t1, 0)
        pltpu.sync_copy(cy.at[pl.ds(0, chunk)], spy.at[pl.ds(o, chunk)])
        plsc.subcore_barrier()

        pltpu.sync_copy(spy, ta.at[pl.ds(0, n_pad)])

        def win2(w, _):
            r0 = t * rows_pt + w * win_rows
            pltpu.sync_copy(srcf.at[pl.ds(r0 * 128, wedges)],
                            sw.at[pl.ds(0, wedges)])
            pltpu.sync_copy(dstr.at[pl.ds(r0, win_rows)],
                            dw.at[pl.ds(0, win_rows)])

            def g(j, _):
                s = pl.ds(j * 16, 16)
                ua[s] = plsc.load_gather(ta, [sw[s]])
                return 0
            lax.fori_loop(0, wedges // 16, g, 0)

            def sc(cc, _):
                pltpu.sync_copy(ua.at[pl.ds(cc * 128, 128)],
                                sps.at[dw.at[cc]], add=True)
                return 0
            lax.fori_loop(0, win_rows, sc, 0)
            return 0
        lax.fori_loop(0, n_win, win2, 0)
        plsc.subcore_barrier()

        pltpu.sync_copy(sps.at[pl.ds(o, chunk)], c0.at[pl.ds(0, chunk)])

        def t3(i, _):
            s = pl.ds(i * 16, 16)
            agg = cdv[s] * (c0[s] + cy[s])
            cyp[s] = cdv[s] * jnp.maximum(agg, 0.0)
            cyq[s] = cdv[s] * jnp.maximum(-agg, 0.0)
            return 0
        lax.fori_loop(0, groups, t3, 0)
        pltpu.sync_copy(cyp.at[pl.ds(0, chunk)], spyp.at[pl.ds(o, chunk)])
        pltpu.sync_copy(cyq.at[pl.ds(0, chunk)], spyq.at[pl.ds(o, chunk)])
        plsc.subcore_barrier()

        pltpu.sync_copy(spyp, ta.at[pl.ds(0, n_pad)])
        pltpu.sync_copy(spyq, tb.at[pl.ds(0, n_pad)])

        def win3(w, _):
            r0 = t * rows_pt + w * win_rows
            pltpu.sync_copy(srcf.at[pl.ds(r0 * 128, wedges)],
                            sw.at[pl.ds(0, wedges)])
            pltpu.sync_copy(dstr.at[pl.ds(r0, win_rows)],
                            dw.at[pl.ds(0, win_rows)])

            def g(j, _):
                s = pl.ds(j * 16, 16)
                idx = sw[s]
                ua[s] = plsc.load_gather(ta, [idx])
                ub[s] = plsc.load_gather(tb, [idx])
                return 0
            lax.fori_loop(0, wedges // 16, g, 0)

            def sc(cc, _):
                pltpu.sync_copy(ua.at[pl.ds(cc * 128, 128)],
                                spp.at[dw.at[cc]], add=True)
                pltpu.sync_copy(ub.at[pl.ds(cc * 128, 128)],
                                spq.at[dw.at[cc]], add=True)
                return 0
            lax.fori_loop(0, win_rows, sc, 0)
            return 0
        lax.fori_loop(0, n_win, win3, 0)
        plsc.subcore_barrier()

        pltpu.sync_copy(spp.at[pl.ds(o, chunk)], c0.at[pl.ds(0, chunk)])
        pltpu.sync_copy(spq.at[pl.ds(o, chunk)], cy.at[pl.ds(0, chunk)])

        def t5(i, _):
            s = pl.ds(i * 16, 16)
            cyp[s] = cdv[s] * (c0[s] + cyp[s])
            cyq[s] = cdv[s] * (cy[s] + cyq[s])
            return 0
        lax.fori_loop(0, groups, t5, 0)
        pltpu.sync_copy(cyp.at[pl.ds(0, chunk)], po.at[pl.ds(o, chunk)])
        pltpu.sync_copy(cyq.at[pl.ds(0, chunk)], qo.at[pl.ds(o, chunk)])
        plsc.subcore_barrier()

    pipeline(src1, dst1, x1h, p1o, q1o,
             s1d, s1s, s1p, s1q, s1y, s1yp, s1yq,
             N1P, ROWS1, WIN1, C1G)
    pipeline(src2, dst2, x2h, p2o, q2o,
             s2d, s2s, s2p, s2q, s2y, s2yp, s2yq,
             N2P, ROWS2, WIN2, C2G)


_sc_mesh = plsc.VectorSubcoreMesh(core_axis_name="c", subcore_axis_name="s",
                                  num_cores=1)

_sc_call = functools.partial(
    pl.kernel,
    out_type=(jax.ShapeDtypeStruct((N1P,), _f32),
              jax.ShapeDtypeStruct((N1P,), _f32),
              jax.ShapeDtypeStruct((N2P,), _f32),
              jax.ShapeDtypeStruct((N2P,), _f32)),
    mesh=_sc_mesh,
    compiler_params=pltpu.CompilerParams(needs_layout_passes=False),
    scratch_types=(
        pltpu.VMEM((WIN1 * 128,), _i32),
        pltpu.VMEM((WIN1, 128), _i32),
        pltpu.VMEM((WIN1 * 128,), _f32),
        pltpu.VMEM((WIN1 * 128,), _f32),
        pltpu.VMEM((N1P,), _f32),
        pltpu.VMEM((N1P,), _f32),
        pltpu.VMEM((C1G,), _f32),
        pltpu.VMEM((C1G,), _f32),
        pltpu.VMEM((C1G,), _f32),
        pltpu.VMEM((C1G,), _f32),
        pltpu.VMEM((C1G,), _f32),
        pltpu.VMEM((C1G,), _f32),
        pltpu.VMEM_SHARED((N1P,), _f32),
        pltpu.VMEM_SHARED((N1P,), _f32),
        pltpu.VMEM_SHARED((N1P,), _f32),
        pltpu.VMEM_SHARED((N1P,), _f32),
        pltpu.VMEM_SHARED((N1P,), _f32),
        pltpu.VMEM_SHARED((N1P,), _f32),
        pltpu.VMEM_SHARED((N1P,), _f32),
        pltpu.VMEM_SHARED((N2P,), _f32),
        pltpu.VMEM_SHARED((N2P,), _f32),
        pltpu.VMEM_SHARED((N2P,), _f32),
        pltpu.VMEM_SHARED((N2P,), _f32),
        pltpu.VMEM_SHARED((N2P,), _f32),
        pltpu.VMEM_SHARED((N2P,), _f32),
        pltpu.VMEM_SHARED((N2P,), _f32),
    ),
)(_sc_body)


NBLK = N1P // 256
NSLOT = SORT_K * NUM_GRAPHS
NSLOTP = 2048


def _tc_keys_body(w1, w2, b2, p1, q1, p2, q2, keys_o, out2_o, ab_o):
    w = w1[...]
    w2v = w2[...]
    b2v = b2[...]
    a = jnp.dot(jnp.maximum(w, 0.0), w2v, preferred_element_type=_f32)
    b = jnp.dot(jnp.maximum(-w, 0.0), w2v, preferred_element_type=_f32)
    ab_o[...] = jnp.concatenate([a, b, jnp.zeros((6, HID), _f32)], axis=0)
    out2 = jnp.maximum(p2[...] * a + q2[...] * b + b2v, 0.0)
    out2_o[...] = out2
    c = out2[N2 - 1:N2, :]
    n2c = jnp.sum(c * c)
    for i in range(NBLK):
        pb = p1[i * 256:(i + 1) * 256, :]
        qb = q1[i * 256:(i + 1) * 256, :]
        o1 = jnp.maximum(pb * a + qb * b + b2v, 0.0)
        n1 = jnp.sum(o1 * o1, axis=1, keepdims=True)
        dc = jnp.sum(o1 * c, axis=1, keepdims=True)
        keys_o[i * 256:(i + 1) * 256, :] = jnp.sqrt(
            jnp.maximum(n1 + n2c - 2.0 * dc, 1e-12))


def _tc_keys(w1, w2, b2, p1c, q1c, p2c, q2c):
    return pl.pallas_call(
        _tc_keys_body,
        out_shape=(jax.ShapeDtypeStruct((N1P, 1), _f32),
                   jax.ShapeDtypeStruct((N2P, HID), _f32),
                   jax.ShapeDtypeStruct((8, HID), _f32)),
    )(w1, w2, b2, p1c, q1c, p2c, q2c)


def _tc_rank_body(keysc, batchc, p1c, q1c, keysr, batchr, sel_o):
    ib = pl.program_id(0)
    iota_s = lax.broadcasted_iota(_i32, (1, NSLOTP), 1)
    iota_c = lax.broadcasted_iota(_i32, (256, 1), 0)
    iota_r = lax.broadcasted_iota(_i32, (1, 256), 1)

    ki = keysc[...]
    bi = batchc[...]
    ii = ib * 256 + iota_c
    blo_i = jnp.min(bi)
    bhi_i = jnp.max(bi)

    acc = jnp.zeros((256, 256), _f32)
    for jb in range(NBLK):
        kj = keysr[jb:jb + 1, :]
        bj = batchr[jb:jb + 1, :]
        jj = jb * 256 + iota_r
        pred = (jnp.min(bj) <= bhi_i) & (jnp.max(bj) >= blo_i)

        def _acc(a, kj=kj, bj=bj, jj=jj):
            same = bi == bj
            beats = (kj > ki) | ((kj == ki) & (jj < ii))
            return a + jnp.where(same & beats, 1.0, 0.0)

        acc = lax.cond(pred, _acc, lambda a: a, acc)

    rank = jnp.dot(acc, jnp.ones((256, 1), _f32),
                   preferred_element_type=_f32).astype(_i32)
    valid = (rank < SORT_K) & (bi < NUM_GRAPHS)
    m = jnp.where(valid, rank * NUM_GRAPHS + bi, NSLOTP - 1)
    onehot = (m == iota_s).astype(_f32)
    lhs = jnp.concatenate([p1c[...], q1c[...], jnp.ones((256, 1), _f32),
                           jnp.zeros((256, 5), _f32)], axis=1)
    upd = lax.dot_general(lhs, onehot, (((0,), (0,)), ((), ())),
                          preferred_element_type=_f32)

    @pl.when(ib == 0)
    def _():
        sel_o[...] = upd

    @pl.when(ib > 0)
    def _():
        sel_o[...] = sel_o[...] + upd


def _tc_rank(keysc, batchc, p1c, q1c, keysr, batchr):
    blk = pl.BlockSpec((256, 1), lambda i: (i, 0))
    return pl.pallas_call(
        _tc_rank_body,
        grid=(NBLK,),
        in_specs=[blk, blk, blk, blk,
                  pl.BlockSpec((NBLK, 256), lambda i: (0, 0)),
                  pl.BlockSpec((NBLK, 256), lambda i: (0, 0))],
        out_specs=pl.BlockSpec((8, NSLOTP), lambda i: (0, 0)),
        out_shape=jax.ShapeDtypeStruct((8, NSLOTP), _f32),
    )(keysc, batchc, p1c, q1c, keysr, batchr)


def _tc_dense_body(ps, qs, vs, out2, ab, b2, fc1w, fc1b, ln1g, ln1b,
                   fc2w, fc2b, ln2g, ln2b, fc3w, fc3b, out_o):
    abv = ab[...]
    a = abv[0:1, :]
    b = abv[1:2, :]
    b2v = b2[...]
    osel = jnp.maximum(ps[...] * a + qs[...] * b + b2v, 0.0)
    o2 = out2[...]
    g = lax.dot_general(osel, o2, (((1,), (1,)), ((), ())),
                        preferred_element_type=_f32)
    ns = jnp.sum(osel * osel, axis=1, keepdims=True)
    n2 = jnp.sum(o2 * o2, axis=1, keepdims=True)
    n2r = lax.dot_general(jnp.ones((1, 1), _f32), n2,
                          (((1,), (1,)), ((), ())),
                          preferred_element_type=_f32)
    drow = jnp.sqrt(jnp.maximum(ns + n2r - 2.0 * g, 1e-12)) * vs[...]

    h = jnp.zeros((NUM_GRAPHS, HID), _f32) + fc1b[...]
    for r in range(SORT_K):
        h = h + jnp.dot(drow[r * NUM_GRAPHS:(r + 1) * NUM_GRAPHS, :],
                        fc1w[r * N2P:(r + 1) * N2P, :],
                        preferred_element_type=_f32)

    def ln(x, gam, bet):
        mu = jnp.mean(x, axis=-1, keepdims=True)
        var = jnp.mean((x - mu) ** 2, axis=-1, keepdims=True)
        return (x - mu) * lax.rsqrt(var + 1e-5) * gam[...] + bet[...]

    h = jnp.maximum(ln(h, ln1g, ln1b), 0.0)
    h = jnp.dot(h, fc2w[...], preferred_element_type=_f32) + fc2b[...]
    h = jnp.maximum(ln(h, ln2g, ln2b), 0.0)
    h = jnp.dot(h, fc3w[...], preferred_element_type=_f32) + fc3b[...]
    out_o[...] = 1.0 / (1.0 + jnp.exp(-h))


def _tc_dense(ps, qs, vs, out2, ab, b2, fc1w, fc1b, ln1g, ln1b,
              fc2w, fc2b, ln2g, ln2b, fc3w, fc3b):
    return pl.pallas_call(
        _tc_dense_body,
        out_shape=jax.ShapeDtypeStruct((NUM_GRAPHS, 1), _f32),
    )(ps, qs, vs, out2, ab, b2, fc1w, fc1b, ln1g, ln1b,
      fc2w, fc2b, ln2g, ln2b, fc3w, fc3b)


@jax.jit
def kernel(x1, edge_index1, batch1, x2, edge_index2, W1, b1, W2, b2,
           fc1_W, fc1_b, ln1_g, ln1_b, fc2_W, fc2_b, ln2_g, ln2_b,
           fc3_W, fc3_b):
    pad1 = jnp.full((E1P - E1,), N1P - 1, _i32)
    src1 = jnp.concatenate([edge_index1[0].astype(_i32), pad1])
    dst1 = jnp.concatenate([edge_index1[1].astype(_i32), pad1]).reshape(R1, 128)
    pad2 = jnp.full((E2P - E2,), N2P - 1, _i32)
    src2 = jnp.concatenate([edge_index2[0].astype(_i32), pad2])
    dst2 = jnp.concatenate([edge_index2[1].astype(_i32), pad2]).reshape(R2, 128)
    x1p = jnp.pad(x1[:, 0], (0, N1P - N1))
    x2p = jnp.pad(x2[:, 0], (0, N2P - N2))
    batchp = jnp.pad(batch1.astype(_i32), (0, N1P - N1),
                     constant_values=NUM_GRAPHS)

    fc1wp = jnp.pad(fc1_W.reshape(SORT_K, N2, HID),
                    ((0, 0), (0, N2P - N2), (0, 0))).reshape(SORT_K * N2P, HID)

    p1h, q1h, p2h, q2h = _sc_call(src1, dst1, x1p, src2, dst2, x2p)

    keysc, out2, ab = _tc_keys(W1, W2, b2[None, :],
                               p1h[:, None], q1h[:, None],
                               p2h[:, None], q2h[:, None])
    keysr = keysc[:, 0].reshape(NBLK, 256)
    sel = _tc_rank(keysc, batchp[:, None], p1h[:, None], q1h[:, None],
                   keysr, batchp.reshape(NBLK, 256))
    ps = sel[0, 0:NSLOT, None]
    qs = sel[1, 0:NSLOT, None]
    vs = sel[2, 0:NSLOT, None]
    out = _tc_dense(ps, qs, vs, out2, ab, b2[None, :], fc1wp,
                    fc1_b[None, :], ln1_g[None, :], ln1_b[None, :],
                    fc2_W, fc2_b[None, :], ln2_g[None, :], ln2_b[None, :],
                    fc3_W, fc3_b[None, :])
    return out
